# baseline (speedup 1.0000x reference)
import jax
import jax.numpy as jnp
from jax import lax
from jax.experimental import pallas as pl
from jax.experimental.pallas import tpu as pltpu, tpu_sc as plsc

L_NUM = 2048
N = L_NUM * L_NUM
ALPHA = 0.8
GAMMA = 0.8

NC, NS, LANES = 2, 16, 16
NW = NC * NS
NRG = L_NUM // 8
RGW = NRG // NW
NH = 2 * RGW
AG = 8 * L_NUM
HAG = AG // 2
QG = 2 * AG
HQG = QG // 2
JB = AG // 128
PLANE = 2 * N
RUN = 2048


def _sc_body(a_hbm, b_hbm, p_hbm, q_hbm, out_hbm,
             a_v, b_v, p_v, q0_v, q1_v, in_sem, out_sem):
    wid = lax.axis_index("s") * NC + lax.axis_index("c")
    rg0 = wid * RGW
    q6 = q_hbm.reshape(2, NRG, 8, 2, 16, 128)
    o6 = out_hbm.reshape(2, NRG, 8, 2, 16, 128)

    def in_descs(hp):
        rg = rg0 + (hp >> 1)
        hh = hp & 1
        bi = hp & 1
        bo = hh * HAG
        ko = rg * AG + hh * HAG
        return [
            pltpu.make_async_copy(a_hbm.at[pl.ds(ko, HAG)],
                                  a_v.at[pl.ds(bo, HAG)], in_sem),
            pltpu.make_async_copy(b_hbm.at[pl.ds(ko, HAG)],
                                  b_v.at[pl.ds(bo, HAG)], in_sem),
            pltpu.make_async_copy(p_hbm.at[pl.ds(ko, HAG)],
                                  p_v.at[pl.ds(bo, HAG)], in_sem),
            pltpu.make_async_copy(q6.at[0, rg, :, hh, :, :],
                                  q0_v.at[bi], in_sem),
            pltpu.make_async_copy(q6.at[1, rg, :, hh, :, :],
                                  q1_v.at[bi], in_sem),
        ]

    def out_descs(hp):
        rg = rg0 + (hp >> 1)
        hh = hp & 1
        bi = hp & 1
        return [
            pltpu.make_async_copy(q0_v.at[bi],
                                  o6.at[0, rg, :, hh, :, :], out_sem),
            pltpu.make_async_copy(q1_v.at[bi],
                                  o6.at[1, rg, :, hh, :, :], out_sem),
        ]

    def compute_part(h, part):
        bo = (h & 1) * HAG
        bi = h & 1

        @plsc.parallel_loop(0, 32)
        def blk(u):
            uu = u + part * 32
            rr = uu >> 3
            jml = uu & 7
            kb = bo + jml * 1024 + rr * 128
            t0 = jml * 2
            for lv in range(8):
                lo = lv * LANES
                a = a_v[pl.ds(kb + lo, LANES)]
                b = b_v[pl.ds(kb + lo, LANES)]
                p = p_v[pl.ds(kb + lo, LANES)]
                q00 = q0_v[bi, rr, t0, pl.ds(lo, LANES)]
                q01 = q0_v[bi, rr, t0 + 1, pl.ds(lo, LANES)]
                q10 = q1_v[bi, rr, t0, pl.ds(lo, LANES)]
                q11 = q1_v[bi, rr, t0 + 1, pl.ds(lo, LANES)]
                ae = a == 0
                be = b == 0
                qb0 = jnp.where(be, q00, q10)
                qb1 = jnp.where(be, q01, q11)
                mx = jnp.maximum(qb0, qb1)
                old = jnp.where(ae,
                                jnp.where(be, q00, q01),
                                jnp.where(be, q10, q11))
                new = old + ALPHA * (p + GAMMA * mx - old)
                q0_v[bi, rr, t0, pl.ds(lo, LANES)] = (
                    jnp.where(ae & be, new, q00))
                q0_v[bi, rr, t0 + 1, pl.ds(lo, LANES)] = (
                    jnp.where(ae & (~be), new, q01))
                q1_v[bi, rr, t0, pl.ds(lo, LANES)] = (
                    jnp.where((~ae) & be, new, q10))
                q1_v[bi, rr, t0 + 1, pl.ds(lo, LANES)] = (
                    jnp.where((~ae) & (~be), new, q11))

    for d in in_descs(0):
        d.start()

    def half(h, carry):
        for d in in_descs(h):
            d.wait()
        compute_part(h, 0)

        @pl.when(h < NH - 1)
        def _():
            @pl.when(h >= 1)
            def _():
                for d in out_descs(h - 1):
                    d.wait()
            for d in in_descs(h + 1):
                d.start()

        compute_part(h, 1)
        for d in out_descs(h):
            d.start()
        return carry

    lax.fori_loop(0, NH, half, 0)
    for d in out_descs(NH - 2):
        d.wait()
    for d in out_descs(NH - 1):
        d.wait()


def _to_tiled_flat(m):
    return m.reshape(NRG, 8, 16, 128).transpose(0, 2, 1, 3).reshape(N)


def kernel(type_t_matrix, type_t1_matrix, Q_tensor, profit_matrix):
    a = _to_tiled_flat(type_t_matrix)
    b = _to_tiled_flat(type_t1_matrix)
    p = _to_tiled_flat(profit_matrix)
    qp = Q_tensor.reshape(NRG * JB, 128, 2, 2).transpose(2, 0, 3, 1)
    qp = qp.reshape(4 * N // 128, 128)
    mesh = plsc.VectorSubcoreMesh(
        core_axis_name="c", subcore_axis_name="s",
        num_cores=NC, num_subcores=NS,
    )
    out = pl.kernel(
        _sc_body,
        out_type=jax.ShapeDtypeStruct((4 * N // 128, 128), jnp.float32),
        mesh=mesh,
        compiler_params=pltpu.CompilerParams(needs_layout_passes=False),
        scratch_types=[
            pltpu.VMEM((2 * HAG,), jnp.int32),
            pltpu.VMEM((2 * HAG,), jnp.int32),
            pltpu.VMEM((2 * HAG,), jnp.float32),
            pltpu.VMEM((2, 8, 16, 128), jnp.float32),
            pltpu.VMEM((2, 8, 16, 128), jnp.float32),
            pltpu.SemaphoreType.DMA,
            pltpu.SemaphoreType.DMA,
        ],
    )(a, b, p, qp)
    out = out.reshape(2, NRG * JB, 2, 128)
    return out.transpose(1, 3, 0, 2).reshape(N, 2, 2)


# speedup vs baseline: 1.2166x; 1.2166x over previous
import jax
import jax.numpy as jnp
from jax import lax
from jax.experimental import pallas as pl
from jax.experimental.pallas import tpu as pltpu, tpu_sc as plsc

L_NUM = 2048
N = L_NUM * L_NUM
ALPHA = 0.8
GAMMA = 0.8

NC, NS, LANES = 2, 16, 16
NW = NC * NS
NRG = L_NUM // 8
RGW = NRG // NW
NH = 2 * RGW
AG = 8 * L_NUM
HAG = AG // 2
QG = 2 * AG
HQG = QG // 2
JB = AG // 128
PLANE = 2 * N
QAG = AG // 4
NU = 4 * RGW


def _sc_body(a_hbm, b_hbm, p_hbm, q_hbm, out_hbm,
             a_v, b_v, p_v, q0_v, q1_v, in_sem, out_sem):
    wid = lax.axis_index("s") * NC + lax.axis_index("c")
    rg0 = wid * RGW
    q6 = q_hbm.reshape(2, NRG, 8, 4, 8, 128)
    o6 = out_hbm.reshape(2, NRG, 8, 4, 8, 128)

    def in_descs(up, bi):
        rg = rg0 + (up >> 2)
        qh = up & 3
        bo = bi * QAG
        ko = rg * AG + qh * QAG
        return [
            pltpu.make_async_copy(a_hbm.at[pl.ds(ko, QAG)],
                                  a_v.at[pl.ds(bo, QAG)], in_sem),
            pltpu.make_async_copy(b_hbm.at[pl.ds(ko, QAG)],
                                  b_v.at[pl.ds(bo, QAG)], in_sem),
            pltpu.make_async_copy(p_hbm.at[pl.ds(ko, QAG)],
                                  p_v.at[pl.ds(bo, QAG)], in_sem),
            pltpu.make_async_copy(q6.at[0, rg, :, qh, :, :],
                                  q0_v.at[bi], in_sem),
            pltpu.make_async_copy(q6.at[1, rg, :, qh, :, :],
                                  q1_v.at[bi], in_sem),
        ]

    def out_descs(up, bi):
        rg = rg0 + (up >> 2)
        qh = up & 3
        return [
            pltpu.make_async_copy(q0_v.at[bi],
                                  o6.at[0, rg, :, qh, :, :], out_sem),
            pltpu.make_async_copy(q1_v.at[bi],
                                  o6.at[1, rg, :, qh, :, :], out_sem),
        ]

    def compute_part(bi, part):
        bo = bi * QAG

        @plsc.parallel_loop(0, 16)
        def blk(u):
            uu = u + part * 16
            rr = uu >> 2
            jml = uu & 3
            kb = bo + jml * 1024 + rr * 128
            t0 = jml * 2
            for lv in range(8):
                lo = lv * LANES
                a = a_v[pl.ds(kb + lo, LANES)]
                b = b_v[pl.ds(kb + lo, LANES)]
                p = p_v[pl.ds(kb + lo, LANES)]
                q00 = q0_v[bi, rr, t0, pl.ds(lo, LANES)]
                q01 = q0_v[bi, rr, t0 + 1, pl.ds(lo, LANES)]
                q10 = q1_v[bi, rr, t0, pl.ds(lo, LANES)]
                q11 = q1_v[bi, rr, t0 + 1, pl.ds(lo, LANES)]
                ae = a == 0
                be = b == 0
                qb0 = jnp.where(be, q00, q10)
                qb1 = jnp.where(be, q01, q11)
                mx = jnp.maximum(qb0, qb1)
                old = jnp.where(ae,
                                jnp.where(be, q00, q01),
                                jnp.where(be, q10, q11))
                new = old + ALPHA * (p + GAMMA * mx - old)
                q0_v[bi, rr, t0, pl.ds(lo, LANES)] = (
                    jnp.where(ae & be, new, q00))
                q0_v[bi, rr, t0 + 1, pl.ds(lo, LANES)] = (
                    jnp.where(ae & (~be), new, q01))
                q1_v[bi, rr, t0, pl.ds(lo, LANES)] = (
                    jnp.where((~ae) & be, new, q10))
                q1_v[bi, rr, t0 + 1, pl.ds(lo, LANES)] = (
                    jnp.where((~ae) & (~be), new, q11))

    for d in in_descs(0, 0):
        d.start()
    for d in in_descs(1, 1):
        d.start()

    def unit(u, carry):
        bi = lax.rem(u, 3)
        for d in in_descs(u, bi):
            d.wait()
        compute_part(bi, 0)

        @pl.when(u + 2 < NU)
        def _():
            bi2 = lax.rem(u + 2, 3)

            @pl.when(u >= 1)
            def _():
                for d in out_descs(u - 1, bi2):
                    d.wait()

            for d in in_descs(u + 2, bi2):
                d.start()

        compute_part(bi, 1)
        for d in out_descs(u, bi):
            d.start()
        return carry

    lax.fori_loop(0, NU, unit, 0)
    for uu in range(NU - 3, NU):
        for d in out_descs(uu, uu % 3):
            d.wait()


def _to_tiled_flat(m):
    return m.reshape(NRG, 8, 16, 128).transpose(0, 2, 1, 3).reshape(N)


def kernel(type_t_matrix, type_t1_matrix, Q_tensor, profit_matrix):
    a = _to_tiled_flat(type_t_matrix)
    b = _to_tiled_flat(type_t1_matrix)
    p = _to_tiled_flat(profit_matrix)
    qp = Q_tensor.reshape(NRG * JB, 128, 2, 2).transpose(2, 0, 3, 1)
    qp = qp.reshape(4 * N // 128, 128)
    mesh = plsc.VectorSubcoreMesh(
        core_axis_name="c", subcore_axis_name="s",
        num_cores=NC, num_subcores=NS,
    )
    out = pl.kernel(
        _sc_body,
        out_type=jax.ShapeDtypeStruct((4 * N // 128, 128), jnp.float32),
        mesh=mesh,
        compiler_params=pltpu.CompilerParams(needs_layout_passes=False),
        scratch_types=[
            pltpu.VMEM((3 * QAG,), jnp.int32),
            pltpu.VMEM((3 * QAG,), jnp.int32),
            pltpu.VMEM((3 * QAG,), jnp.float32),
            pltpu.VMEM((3, 8, 8, 128), jnp.float32),
            pltpu.VMEM((3, 8, 8, 128), jnp.float32),
            pltpu.SemaphoreType.DMA,
            pltpu.SemaphoreType.DMA,
        ],
    )(a, b, p, qp)
    out = out.reshape(2, NRG * JB, 2, 128)
    return out.transpose(1, 3, 0, 2).reshape(N, 2, 2)
